# baseline (speedup 1.0000x reference)
import functools

import jax
import jax.numpy as jnp
from jax import lax
from jax.experimental import pallas as pl
from jax.experimental.pallas import tpu as pltpu
from jax.experimental.pallas import tpu_sc as plsc

_NMONTH, _NTID, _NWEEK, _NHOL, _NDT = 12, 288, 7, 2, 3
_NCOMBO = _NMONTH * _NWEEK * _NHOL * _NDT
_NROWS = 792
_HID = 128

_NC, _NS = 2, 16
_NW = _NC * _NS
_CHUNK = 80
_SB = 8
_NSLOT = 4
_NREP = 1536


_REPBLK = 16


def _build_table_body(month_ref, tid_ref, week_ref, hol_ref, dt_ref, w_ref,
                      b_ref, out_ref, tab_scratch):
    @pl.when(pl.program_id(0) == 0)
    def _compute():
        _fill_table(month_ref, tid_ref, week_ref, hol_ref, dt_ref, w_ref,
                    b_ref, tab_scratch)

    for r in range(_REPBLK):
        out_ref[r * _NROWS:(r + 1) * _NROWS, :] = tab_scratch[...]


def _fill_table(month_ref, tid_ref, week_ref, hol_ref, dt_ref, w_ref,
                b_ref, out_ref):
    c = lax.broadcasted_iota(jnp.int32, (_NCOMBO, 1), 0)
    m = c // (_NWEEK * _NHOL * _NDT)
    w = (c // (_NHOL * _NDT)) % _NWEEK
    h = (c // _NDT) % _NHOL
    d = c % _NDT

    def onehot(idx, n):
        return (idx == lax.broadcasted_iota(jnp.int32, (_NCOMBO, n), 1)
                ).astype(jnp.float32)

    wmat = w_ref[...]
    proj_m = jnp.dot(month_ref[...], wmat[0:4, :], preferred_element_type=jnp.float32)
    proj_w = jnp.dot(week_ref[...], wmat[8:12, :], preferred_element_type=jnp.float32)
    proj_h = jnp.dot(hol_ref[...], wmat[12:14, :], preferred_element_type=jnp.float32)
    proj_d = jnp.dot(dt_ref[...], wmat[14:16, :], preferred_element_type=jnp.float32)
    combo = (jnp.dot(onehot(m, _NMONTH), proj_m, preferred_element_type=jnp.float32)
             + jnp.dot(onehot(w, _NWEEK), proj_w, preferred_element_type=jnp.float32)
             + jnp.dot(onehot(h, _NHOL), proj_h, preferred_element_type=jnp.float32)
             + jnp.dot(onehot(d, _NDT), proj_d, preferred_element_type=jnp.float32)
             + b_ref[...])
    tid_proj = jnp.dot(tid_ref[...], wmat[4:8, :],
                       preferred_element_type=jnp.float32)
    out_ref[0:_NCOMBO, :] = combo
    out_ref[_NCOMBO:_NCOMBO + _NTID, :] = tid_proj


def _build_table(month_tab, tid_tab, week_tab, holiday_tab, date_type_tab,
                 fuse_W, fuse_b):
    full = lambda s: pl.BlockSpec(s, lambda i: (0,) * len(s))
    return pl.pallas_call(
        _build_table_body,
        grid=(_NREP // _REPBLK,),
        in_specs=[
            full((_NMONTH, 4)), full((_NTID, 4)), full((_NWEEK, 4)),
            full((_NHOL, 2)), full((_NDT, 2)), full((16, _HID)),
            full((1, _HID)),
        ],
        out_specs=pl.BlockSpec((_REPBLK * _NROWS, _HID), lambda i: (i, 0)),
        out_shape=jax.ShapeDtypeStruct((_NREP * _NROWS, _HID), jnp.float32),
        scratch_shapes=[pltpu.VMEM((_NROWS, _HID), jnp.float32)],
    )(month_tab, tid_tab, week_tab, holiday_tab, date_type_tab, fuse_W,
      fuse_b.reshape(1, _HID))


def _make_sc_gather(n_tokens):
    n_per_w = n_tokens // _NW
    n_chunks = n_per_w // _CHUNK
    n_sb = n_chunks // _SB
    mesh = plsc.VectorSubcoreMesh(core_axis_name="c", subcore_axis_name="s")

    @functools.partial(
        pl.kernel,
        mesh=mesh,
        out_type=jax.ShapeDtypeStruct((n_tokens, _HID), jnp.float32),
        scratch_types=[
            pltpu.VMEM((5 * _SB * _CHUNK,), jnp.int32),
            [pltpu.VMEM((_CHUNK, _HID), jnp.float32) for _ in range(_NSLOT)],
            [pltpu.VMEM((_CHUNK, _HID), jnp.float32) for _ in range(_NSLOT)],
            [pltpu.VMEM((_CHUNK,), jnp.int32) for _ in range(_NSLOT)],
            [pltpu.VMEM((_CHUNK,), jnp.int32) for _ in range(_NSLOT)],
            pltpu.SemaphoreType.DMA((_NSLOT,)),
            pltpu.SemaphoreType.DMA((_NSLOT,)),
        ],
        compiler_params=pltpu.CompilerParams(needs_layout_passes=False),
    )
    def sc_gather(tf_hbm, tab_hbm, out_hbm, tf_v, buf_a, buf_b,
                  cidx, tidx, gsem, ssem):
        sid = lax.axis_index("s")
        wid = sid * _NC + lax.axis_index("c")
        tile_base = wid * n_per_w

        def stage_tf(sb):
            pltpu.sync_copy(tf_hbm.at[wid * n_sb + sb], tf_v)

        def issue(g, j, s, guard):
            gbase = g * _CHUNK + wid * (_NREP // _NW)
            for i in range(_CHUNK // 16):
                pos = i * 16 + lax.iota(jnp.int32, 16)
                lane5 = (j * _CHUNK + pos) * 5
                m = plsc.load_gather(tf_v, [lane5])
                t = plsc.load_gather(tf_v, [lane5 + 1])
                w = plsc.load_gather(tf_v, [lane5 + 2])
                h = plsc.load_gather(tf_v, [lane5 + 3])
                d = plsc.load_gather(tf_v, [lane5 + 4])
                rep = ((pos + gbase) % _NREP) * _NROWS
                cidx[s][pl.ds(i * 16, 16)] = (
                    rep + (((m - 1) * _NWEEK + w) * _NHOL + h) * _NDT + d)
                tidx[s][pl.ds(i * 16, 16)] = rep + t + _NCOMBO
            if guard:
                pltpu.make_async_copy(
                    buf_a[s], out_hbm.at[pl.ds(tile_base, _CHUNK)],
                    ssem.at[s]).wait()
            pltpu.async_copy(tab_hbm.at[cidx[s]], buf_a[s], gsem.at[s])
            pltpu.async_copy(tab_hbm.at[tidx[s]], buf_b[s], gsem.at[s])

        def finish(g, s):
            pltpu.make_async_copy(tab_hbm.at[cidx[s]], buf_a[s],
                                  gsem.at[s]).wait()
            pltpu.make_async_copy(tab_hbm.at[tidx[s]], buf_b[s],
                                  gsem.at[s]).wait()

            def addj(jj, carry):
                for kk in range(_HID // 16):
                    buf_a[s][jj, pl.ds(kk * 16, 16)] = (
                        buf_a[s][jj, pl.ds(kk * 16, 16)]
                        + buf_b[s][jj, pl.ds(kk * 16, 16)])
                return carry

            lax.fori_loop(0, _CHUNK, addj, 0)
            pltpu.async_copy(
                buf_a[s], out_hbm.at[pl.ds(tile_base + g * _CHUNK, _CHUNK)],
                ssem.at[s])

        stage_tf(0)
        for j in range(_SB):
            issue(j, j, j % _NSLOT, guard=j >= _NSLOT)
            if j >= 2:
                finish(j - 2, (j - 2) % _NSLOT)

        def sb_body(sb, carry):
            g0 = sb * _SB
            stage_tf(sb)
            for j in range(_SB):
                issue(g0 + j, j, j % _NSLOT, guard=True)
                finish(g0 + j - 2, (j - 2) % _NSLOT)
            return carry

        lax.fori_loop(1, n_sb, sb_body, 0)
        last = n_chunks - 1
        finish(last - 1, (last - 1) % _NSLOT)
        finish(last, last % _NSLOT)
        for s in range(_NSLOT):
            pltpu.make_async_copy(
                buf_a[s], out_hbm.at[pl.ds(tile_base, _CHUNK)],
                ssem.at[s]).wait()

    return sc_gather


def kernel(time_features, month_tab, tid_tab, week_tab, holiday_tab,
           date_type_tab, fuse_W, fuse_b):
    b, l, _ = time_features.shape
    n_tokens = b * l
    table = _build_table(month_tab, tid_tab, week_tab, holiday_tab,
                         date_type_tab, fuse_W, fuse_b)
    n_rows_tf = n_tokens // (_SB * _CHUNK)
    flat = _make_sc_gather(n_tokens)(
        time_features.reshape(n_rows_tf, 5 * _SB * _CHUNK), table)
    return flat.reshape(b, l, _HID)


# speedup vs baseline: 1.0591x; 1.0591x over previous
import functools

import jax
import jax.numpy as jnp
from jax import lax
from jax.experimental import pallas as pl
from jax.experimental.pallas import tpu as pltpu
from jax.experimental.pallas import tpu_sc as plsc

_NMONTH, _NTID, _NWEEK, _NHOL, _NDT = 12, 288, 7, 2, 3
_NCOMBO = _NMONTH * _NWEEK * _NHOL * _NDT
_NROWS = 792
_HID = 128

_NC, _NS = 2, 16
_NW = _NC * _NS
_CHUNK = 80
_SB = 8
_NSLOT = 4
_NREP = 1024


_REPBLK = 16


def _build_table_body(month_ref, tid_ref, week_ref, hol_ref, dt_ref, w_ref,
                      b_ref, out_ref, tab_scratch):
    @pl.when(pl.program_id(0) == 0)
    def _compute():
        _fill_table(month_ref, tid_ref, week_ref, hol_ref, dt_ref, w_ref,
                    b_ref, tab_scratch)

    for r in range(_REPBLK):
        out_ref[r * _NROWS:(r + 1) * _NROWS, :] = tab_scratch[...]


def _fill_table(month_ref, tid_ref, week_ref, hol_ref, dt_ref, w_ref,
                b_ref, out_ref):
    c = lax.broadcasted_iota(jnp.int32, (_NCOMBO, 1), 0)
    m = c // (_NWEEK * _NHOL * _NDT)
    w = (c // (_NHOL * _NDT)) % _NWEEK
    h = (c // _NDT) % _NHOL
    d = c % _NDT

    def onehot(idx, n):
        return (idx == lax.broadcasted_iota(jnp.int32, (_NCOMBO, n), 1)
                ).astype(jnp.float32)

    wmat = w_ref[...]
    proj_m = jnp.dot(month_ref[...], wmat[0:4, :], preferred_element_type=jnp.float32)
    proj_w = jnp.dot(week_ref[...], wmat[8:12, :], preferred_element_type=jnp.float32)
    proj_h = jnp.dot(hol_ref[...], wmat[12:14, :], preferred_element_type=jnp.float32)
    proj_d = jnp.dot(dt_ref[...], wmat[14:16, :], preferred_element_type=jnp.float32)
    combo = (jnp.dot(onehot(m, _NMONTH), proj_m, preferred_element_type=jnp.float32)
             + jnp.dot(onehot(w, _NWEEK), proj_w, preferred_element_type=jnp.float32)
             + jnp.dot(onehot(h, _NHOL), proj_h, preferred_element_type=jnp.float32)
             + jnp.dot(onehot(d, _NDT), proj_d, preferred_element_type=jnp.float32)
             + b_ref[...])
    tid_proj = jnp.dot(tid_ref[...], wmat[4:8, :],
                       preferred_element_type=jnp.float32)
    out_ref[0:_NCOMBO, :] = combo
    out_ref[_NCOMBO:_NCOMBO + _NTID, :] = tid_proj


def _build_table(month_tab, tid_tab, week_tab, holiday_tab, date_type_tab,
                 fuse_W, fuse_b):
    full = lambda s: pl.BlockSpec(s, lambda i: (0,) * len(s))
    return pl.pallas_call(
        _build_table_body,
        grid=(_NREP // _REPBLK,),
        in_specs=[
            full((_NMONTH, 4)), full((_NTID, 4)), full((_NWEEK, 4)),
            full((_NHOL, 2)), full((_NDT, 2)), full((16, _HID)),
            full((1, _HID)),
        ],
        out_specs=pl.BlockSpec((_REPBLK * _NROWS, _HID), lambda i: (i, 0)),
        out_shape=jax.ShapeDtypeStruct((_NREP * _NROWS, _HID), jnp.float32),
        scratch_shapes=[pltpu.VMEM((_NROWS, _HID), jnp.float32)],
    )(month_tab, tid_tab, week_tab, holiday_tab, date_type_tab, fuse_W,
      fuse_b.reshape(1, _HID))


def _make_sc_gather(n_tokens):
    n_per_w = n_tokens // _NW
    n_chunks = n_per_w // _CHUNK
    n_sb = n_chunks // _SB
    mesh = plsc.VectorSubcoreMesh(core_axis_name="c", subcore_axis_name="s")

    @functools.partial(
        pl.kernel,
        mesh=mesh,
        out_type=jax.ShapeDtypeStruct((n_tokens, _HID), jnp.float32),
        scratch_types=[
            pltpu.VMEM((5 * _SB * _CHUNK,), jnp.int32),
            [pltpu.VMEM((_CHUNK, _HID), jnp.float32) for _ in range(_NSLOT)],
            [pltpu.VMEM((_CHUNK, _HID), jnp.float32) for _ in range(_NSLOT)],
            [pltpu.VMEM((_CHUNK,), jnp.int32) for _ in range(_NSLOT)],
            [pltpu.VMEM((_CHUNK,), jnp.int32) for _ in range(_NSLOT)],
            pltpu.SemaphoreType.DMA((_NSLOT,)),
            pltpu.SemaphoreType.DMA((_NSLOT,)),
        ],
        compiler_params=pltpu.CompilerParams(needs_layout_passes=False),
    )
    def sc_gather(tf_hbm, tab_hbm, out_hbm, tf_v, buf_a, buf_b,
                  cidx, tidx, gsem, ssem):
        sid = lax.axis_index("s")
        wid = sid * _NC + lax.axis_index("c")
        tile_base = wid * n_per_w

        def stage_tf(sb):
            pltpu.sync_copy(tf_hbm.at[wid * n_sb + sb], tf_v)

        def issue(g, j, s, guard):
            gbase = g * _CHUNK + wid * (_NREP // _NW)
            for i in range(_CHUNK // 16):
                pos = i * 16 + lax.iota(jnp.int32, 16)
                lane5 = (j * _CHUNK + pos) * 5
                m = plsc.load_gather(tf_v, [lane5])
                t = plsc.load_gather(tf_v, [lane5 + 1])
                w = plsc.load_gather(tf_v, [lane5 + 2])
                h = plsc.load_gather(tf_v, [lane5 + 3])
                d = plsc.load_gather(tf_v, [lane5 + 4])
                rep = ((pos + gbase) % _NREP) * _NROWS
                cidx[s][pl.ds(i * 16, 16)] = (
                    rep + (((m - 1) * _NWEEK + w) * _NHOL + h) * _NDT + d)
                tidx[s][pl.ds(i * 16, 16)] = rep + t + _NCOMBO
            if guard:
                pltpu.make_async_copy(
                    buf_a[s], out_hbm.at[pl.ds(tile_base, _CHUNK)],
                    ssem.at[s]).wait()
            pltpu.async_copy(tab_hbm.at[cidx[s]], buf_a[s], gsem.at[s])
            pltpu.async_copy(tab_hbm.at[tidx[s]], buf_b[s], gsem.at[s])

        def finish(g, s):
            pltpu.make_async_copy(tab_hbm.at[cidx[s]], buf_a[s],
                                  gsem.at[s]).wait()
            pltpu.make_async_copy(tab_hbm.at[tidx[s]], buf_b[s],
                                  gsem.at[s]).wait()

            def addj(jj, carry):
                for kk in range(_HID // 16):
                    buf_a[s][jj, pl.ds(kk * 16, 16)] = (
                        buf_a[s][jj, pl.ds(kk * 16, 16)]
                        + buf_b[s][jj, pl.ds(kk * 16, 16)])
                return carry

            lax.fori_loop(0, _CHUNK, addj, 0)
            pltpu.async_copy(
                buf_a[s], out_hbm.at[pl.ds(tile_base + g * _CHUNK, _CHUNK)],
                ssem.at[s])

        stage_tf(0)
        for j in range(_SB):
            issue(j, j, j % _NSLOT, guard=j >= _NSLOT)
            if j >= 2:
                finish(j - 2, (j - 2) % _NSLOT)

        def sb_body(sb, carry):
            g0 = sb * _SB
            stage_tf(sb)
            for j in range(_SB):
                issue(g0 + j, j, j % _NSLOT, guard=True)
                finish(g0 + j - 2, (j - 2) % _NSLOT)
            return carry

        lax.fori_loop(1, n_sb, sb_body, 0)
        last = n_chunks - 1
        finish(last - 1, (last - 1) % _NSLOT)
        finish(last, last % _NSLOT)
        for s in range(_NSLOT):
            pltpu.make_async_copy(
                buf_a[s], out_hbm.at[pl.ds(tile_base, _CHUNK)],
                ssem.at[s]).wait()

    return sc_gather


def kernel(time_features, month_tab, tid_tab, week_tab, holiday_tab,
           date_type_tab, fuse_W, fuse_b):
    b, l, _ = time_features.shape
    n_tokens = b * l
    table = _build_table(month_tab, tid_tab, week_tab, holiday_tab,
                         date_type_tab, fuse_W, fuse_b)
    n_rows_tf = n_tokens // (_SB * _CHUNK)
    flat = _make_sc_gather(n_tokens)(
        time_features.reshape(n_rows_tf, 5 * _SB * _CHUNK), table)
    return flat.reshape(b, l, _HID)
